# baseline (speedup 1.0000x reference)
import dataclasses
import functools

import jax
import jax.numpy as jnp
from jax import lax
from jax.experimental import pallas as pl
from jax.experimental.pallas import tpu as pltpu
from jax.experimental.pallas import tpu_sc as plsc

_VPAD = 128
_NC, _NS = 2, 16
_NW = _NC * _NS
_LANES = 16


@functools.partial(jax.jit, static_argnames=("n",))
def _sc_call(idx, scores, w1, w2, prior, bias, thresh, *, n):
    per_w = n // _NW
    mesh = plsc.VectorSubcoreMesh(core_axis_name="c", subcore_axis_name="s")
    cp = pltpu.CompilerParams()
    if "needs_layout_passes" in pltpu.CompilerParams.__dataclass_fields__:
        cp = dataclasses.replace(cp, needs_layout_passes=False)

    @functools.partial(
        pl.kernel,
        out_type=[
            jax.ShapeDtypeStruct((n,), jnp.float32),
            jax.ShapeDtypeStruct((n,), jnp.int32),
        ],
        mesh=mesh,
        scratch_types=[
            pltpu.VMEM((per_w,), jnp.int32),
            pltpu.VMEM((per_w,), jnp.float32),
            pltpu.VMEM((per_w,), jnp.float32),
            pltpu.VMEM((per_w,), jnp.int32),
            pltpu.VMEM((_VPAD,), jnp.float32),
            pltpu.VMEM((_VPAD,), jnp.float32),
            pltpu.VMEM((_VPAD,), jnp.float32),
            pltpu.VMEM((_VPAD,), jnp.float32),
            pltpu.VMEM((_VPAD,), jnp.float32),
        ],
        compiler_params=cp,
    )
    def body(idx_hbm, s_hbm, w1_hbm, w2_hbm, pr_hbm, bi_hbm, th_hbm,
             cal_hbm, mask_hbm,
             idx_v, s_v, cal_v, m_v, w1_v, c_v, pr_v, bi_v, th_v):
        wid = lax.axis_index("s") * _NC + lax.axis_index("c")
        base = wid * per_w
        pltpu.sync_copy(idx_hbm.at[pl.ds(base, per_w)], idx_v)
        pltpu.sync_copy(s_hbm.at[pl.ds(base, per_w)], s_v)
        pltpu.sync_copy(w1_hbm, w1_v)
        pltpu.sync_copy(w2_hbm, c_v)
        pltpu.sync_copy(pr_hbm, pr_v)
        pltpu.sync_copy(bi_hbm, bi_v)
        pltpu.sync_copy(th_hbm, th_v)

        @pl.loop(0, _VPAD, step=_LANES)
        def _(i):
            sl = pl.ds(i, _LANES)
            c_v[sl] = c_v[sl] * pr_v[sl] + bi_v[sl]

        @pl.loop(0, per_w, step=_LANES, unroll=8)
        def _(i):
            sl = pl.ds(i, _LANES)
            ids = idx_v[sl]
            a = plsc.load_gather(w1_v, [ids])
            c = plsc.load_gather(c_v, [ids])
            th = plsc.load_gather(th_v, [ids])
            logits = a * s_v[sl] + c
            cal = 1.0 / (1.0 + jnp.exp(-logits))
            cal_v[sl] = cal
            m_v[sl] = jnp.where(cal > th, jnp.int32(1), jnp.int32(0))

        pltpu.sync_copy(cal_v, cal_hbm.at[pl.ds(base, per_w)])
        pltpu.sync_copy(m_v, mask_hbm.at[pl.ds(base, per_w)])

    return body(idx, scores, w1, w2, prior, bias, thresh)


def kernel(type_ids, scores, prior, weights, bias, threshold):
    b, l = type_ids.shape
    n = b * l
    v = prior.shape[0]
    pad = _VPAD - v
    idx = type_ids.reshape(n).astype(jnp.int32)
    s = scores.reshape(n)
    w1 = jnp.pad(weights[:, 0], (0, pad))
    w2 = jnp.pad(weights[:, 1], (0, pad))
    pr = jnp.pad(prior, (0, pad))
    bi = jnp.pad(bias, (0, pad))
    th = jnp.pad(threshold, (0, pad))
    cal, mask = _sc_call(idx, s, w1, w2, pr, bi, th, n=n)
    return cal.reshape(b, l), mask.astype(jnp.bool_).reshape(b, l)


# speedup vs baseline: 1.3585x; 1.3585x over previous
import dataclasses
import functools

import jax
import jax.numpy as jnp
from jax import lax
from jax.experimental import pallas as pl
from jax.experimental.pallas import tpu as pltpu
from jax.experimental.pallas import tpu_sc as plsc

_VPAD = 128
_NC, _NS = 2, 16
_NW = _NC * _NS
_LANES = 16


@functools.partial(jax.jit, static_argnames=("n",))
def _sc_call(idx, scores, w1, w2, prior, bias, thresh, *, n):
    per_w = n // _NW
    mesh = plsc.VectorSubcoreMesh(core_axis_name="c", subcore_axis_name="s")
    cp = pltpu.CompilerParams()
    if "needs_layout_passes" in pltpu.CompilerParams.__dataclass_fields__:
        cp = dataclasses.replace(cp, needs_layout_passes=False)

    @functools.partial(
        pl.kernel,
        out_type=[
            jax.ShapeDtypeStruct((n,), jnp.float32),
            jax.ShapeDtypeStruct((n,), jnp.int32),
        ],
        mesh=mesh,
        scratch_types=[
            pltpu.VMEM((per_w,), jnp.int32),
            pltpu.VMEM((per_w,), jnp.float32),
            pltpu.VMEM((per_w,), jnp.float32),
            pltpu.VMEM((per_w,), jnp.int32),
            pltpu.VMEM((_VPAD,), jnp.float32),
            pltpu.VMEM((_VPAD,), jnp.float32),
            pltpu.VMEM((_VPAD,), jnp.float32),
            pltpu.VMEM((_VPAD,), jnp.float32),
            pltpu.VMEM((_VPAD,), jnp.float32),
        ],
        compiler_params=cp,
    )
    def body(idx_hbm, s_hbm, w1_hbm, w2_hbm, pr_hbm, bi_hbm, th_hbm,
             cal_hbm, mask_hbm,
             idx_v, s_v, cal_v, m_v, w1_v, c_v, pr_v, bi_v, th_v):
        wid = lax.axis_index("s") * _NC + lax.axis_index("c")
        base = wid * per_w
        pltpu.sync_copy(idx_hbm.at[pl.ds(base, per_w)], idx_v)
        pltpu.sync_copy(s_hbm.at[pl.ds(base, per_w)], s_v)
        pltpu.sync_copy(w1_hbm, w1_v)
        pltpu.sync_copy(w2_hbm, c_v)
        pltpu.sync_copy(pr_hbm, pr_v)
        pltpu.sync_copy(bi_hbm, bi_v)
        pltpu.sync_copy(th_hbm, th_v)

        @pl.loop(0, _VPAD, step=_LANES)
        def _(i):
            sl = pl.ds(i, _LANES)
            c_v[sl] = -(c_v[sl] * pr_v[sl] + bi_v[sl])
            w1_v[sl] = -w1_v[sl]

        @plsc.parallel_loop(0, per_w, step=_LANES, unroll=8)
        def _(i):
            sl = pl.ds(i, _LANES)
            ids = idx_v[sl]
            na = plsc.load_gather(w1_v, [ids])
            nc = plsc.load_gather(c_v, [ids])
            th = plsc.load_gather(th_v, [ids])
            e = jnp.exp(na * s_v[sl] + nc)
            cal = 1.0 / (1.0 + e)
            cal_v[sl] = cal
            m_v[sl] = jnp.where(cal > th, jnp.int32(1), jnp.int32(0))

        pltpu.sync_copy(cal_v, cal_hbm.at[pl.ds(base, per_w)])
        pltpu.sync_copy(m_v, mask_hbm.at[pl.ds(base, per_w)])

    return body(idx, scores, w1, w2, prior, bias, thresh)


def kernel(type_ids, scores, prior, weights, bias, threshold):
    b, l = type_ids.shape
    n = b * l
    v = prior.shape[0]
    pad = _VPAD - v
    idx = type_ids.reshape(n).astype(jnp.int32)
    s = scores.reshape(n)
    w1 = jnp.pad(weights[:, 0], (0, pad))
    w2 = jnp.pad(weights[:, 1], (0, pad))
    pr = jnp.pad(prior, (0, pad))
    bi = jnp.pad(bias, (0, pad))
    th = jnp.pad(threshold, (0, pad))
    cal, mask = _sc_call(idx, s, w1, w2, pr, bi, th, n=n)
    return cal.reshape(b, l), mask.astype(jnp.bool_).reshape(b, l)
